# baseline (speedup 1.0000x reference)
import jax
import jax.numpy as jnp
from jax import lax
from jax.experimental import pallas as pl
from jax.experimental.pallas import tpu as pltpu
from jax.experimental.pallas import tpu_sc as plsc

N_USERS = 50000
N_ITEMS = 50000
N_NODES = 100000
EMB = 32
LAYERS = 3
BATCH = 4096

NC = 2
NS = 16
NW = NC * NS
G = 128
SB = 4
HALF = 50000
ACC_ROWS = 50176
ZPT = ACC_ROWS // NS

TE = 50000
PB = 8
PNB = TE // (PB * G)
TAILE = TE - PNB * PB * G
FLCAP = 1280
CAP = TE + 1024

_mesh = plsc.VectorSubcoreMesh(core_axis_name="c", subcore_axis_name="s")

_GDN = lax.GatherDimensionNumbers(
    offset_dims=(), collapsed_slice_dims=(0,), start_index_map=(0,))


def _bcast_lane(v16, e):
    idx = jnp.full((16, 1), e, jnp.int32)
    return lax.gather(v16, idx, _GDN, (1,),
                      mode=lax.GatherScatterMode.PROMISE_IN_BOUNDS)


def _partition_body(cols, rows, vals, pedge, cnt,
                    icol, irow, ival, bufs, cbuf, lsem, fsem):
    c = lax.axis_index("c")
    s = lax.axis_index("s")
    w = s * NC + c
    ebase = w * TE

    (bc0, br0, bv0), (bc1, br1, bv1) = bufs

    def fire_lin(b, slot):
        be = ebase + b * PB * G
        pltpu.async_copy(cols.at[pl.ds(be, PB * G)], icol.at[slot], lsem)
        pltpu.async_copy(rows.at[pl.ds(be, PB * G)], irow.at[slot], lsem)
        pltpu.async_copy(vals.at[pl.ds(be, PB * G)], ival.at[slot], lsem)

    def wait_lin(slot):
        pltpu.make_async_copy(cols.at[pl.ds(0, PB * G)], icol.at[slot], lsem).wait()
        pltpu.make_async_copy(rows.at[pl.ds(0, PB * G)], irow.at[slot], lsem).wait()
        pltpu.make_async_copy(vals.at[pl.ds(0, PB * G)], ival.at[slot], lsem).wait()

    fire_lin(0, 0)

    def flush(h, bc, br, bv, cur, hb):
        nf = lax.shift_right_logical(cur, 7)
        hb = pl.multiple_of(hb, G)

        def fcopy(g, carry):
            o = pl.multiple_of(g * G, G)
            pltpu.async_copy(bc.at[pl.ds(o, G)], pedge.at[h, w, 0, pl.ds(hb + o, G)], fsem)
            pltpu.async_copy(br.at[pl.ds(o, G)], pedge.at[h, w, 1, pl.ds(hb + o, G)], fsem)
            pltpu.async_copy(bv.at[pl.ds(o, G)], pedge.at[h, w, 2, pl.ds(hb + o, G)], fsem)
            return carry

        lax.fori_loop(0, nf, fcopy, 0)

        def fdrain(g, carry):
            o = pl.multiple_of(g * G, G)
            pltpu.make_async_copy(bc.at[pl.ds(o, G)], pedge.at[h, w, 0, pl.ds(hb + o, G)], fsem).wait()
            pltpu.make_async_copy(br.at[pl.ds(o, G)], pedge.at[h, w, 1, pl.ds(hb + o, G)], fsem).wait()
            pltpu.make_async_copy(bv.at[pl.ds(o, G)], pedge.at[h, w, 2, pl.ds(hb + o, G)], fsem).wait()
            return carry

        lax.fori_loop(0, nf, fdrain, 0)

        fb = nf * G
        for t in range(G // 16):
            tc = bc[pl.ds(fb + t * 16, 16)]
            tr = br[pl.ds(fb + t * 16, 16)]
            tv = bv[pl.ds(fb + t * 16, 16)]
            bc[pl.ds(t * 16, 16)] = tc
            br[pl.ds(t * 16, 16)] = tr
            bv[pl.ds(t * 16, 16)] = tv
        return cur - fb, hb + fb

    def route(slot, q, cur0, cur1):
        c16 = icol[slot, pl.ds(q * 16, 16)]
        r16 = irow[slot, pl.ds(q * 16, 16)]
        v16 = plsc.bitcast(ival[slot, pl.ds(q * 16, 16)], jnp.int32)
        m0 = r16 < HALF
        m1 = jnp.logical_not(m0)
        plsc.store_compressed(bc0.at[pl.ds(cur0, 16)], c16, mask=m0)
        plsc.store_compressed(br0.at[pl.ds(cur0, 16)], r16, mask=m0)
        plsc.store_compressed(bv0.at[pl.ds(cur0, 16)], v16, mask=m0)
        plsc.store_compressed(bc1.at[pl.ds(cur1, 16)], c16, mask=m1)
        plsc.store_compressed(br1.at[pl.ds(cur1, 16)], r16, mask=m1)
        plsc.store_compressed(bv1.at[pl.ds(cur1, 16)], v16, mask=m1)
        n0 = plsc.all_reduce_population_count(m0)[0]
        return cur0 + n0, cur1 + (16 - n0)

    def block(b, carry):
        cur0, cur1, hb0, hb1 = carry
        slot = lax.rem(b, 2)
        wait_lin(slot)

        @pl.when(b + 1 < PNB)
        def _next():
            fire_lin(b + 1, 1 - slot)

        for q in range(PB * G // 16):
            cur0, cur1 = route(slot, q, cur0, cur1)

        cur0, hb0 = flush(0, bc0, br0, bv0, cur0, hb0)
        cur1, hb1 = flush(1, bc1, br1, bv1, cur1, hb1)
        return cur0, cur1, hb0, hb1

    cur0, cur1, hb0, hb1 = lax.fori_loop(
        0, PNB, block, (jnp.int32(0), jnp.int32(0), jnp.int32(0), jnp.int32(0)))

    pltpu.sync_copy(cols.at[pl.ds(ebase + PNB * PB * G, TAILE)],
                    icol.at[0, pl.ds(0, TAILE)])
    pltpu.sync_copy(rows.at[pl.ds(ebase + PNB * PB * G, TAILE)],
                    irow.at[0, pl.ds(0, TAILE)])
    pltpu.sync_copy(vals.at[pl.ds(ebase + PNB * PB * G, TAILE)],
                    ival.at[0, pl.ds(0, TAILE)])
    for q in range(TAILE // 16):
        cur0, cur1 = route(0, q, cur0, cur1)
    cur0, hb0 = flush(0, bc0, br0, bv0, cur0, hb0)
    cur1, hb1 = flush(1, bc1, br1, bv1, cur1, hb1)

    zi = jnp.zeros((16,), jnp.int32)
    for (bc, br, bv, cur) in ((bc0, br0, bv0, cur0), (bc1, br1, bv1, cur1)):
        for t in range(512 // 16):
            bc[pl.ds(cur + t * 16, 16)] = zi
            br[pl.ds(cur + t * 16, 16)] = zi
            bv[pl.ds(cur + t * 16, 16)] = zi
    for h, (bc, br, bv, cur, hb) in enumerate(
            ((bc0, br0, bv0, cur0, hb0), (bc1, br1, bv1, cur1, hb1))):
        total = hb + cur
        totp = jnp.bitwise_and(total + 511, -512)
        curp = totp - hb
        flush(h, bc, br, bv, curp, hb)
        ng = lax.shift_right_logical(totp, 7)
        cbuf[pl.ds(0, 16)] = jnp.full((16,), ng, jnp.int32)
        pltpu.sync_copy(cbuf, cnt.at[h, w])


_partition = pl.kernel(
    _partition_body,
    out_type=(
        jax.ShapeDtypeStruct((2, NW, 3, CAP), jnp.int32),
        jax.ShapeDtypeStruct((2, NW, 16), jnp.int32),
    ),
    mesh=_mesh,
    compiler_params=pltpu.CompilerParams(use_tc_tiling_on_sc=False,
                                         needs_layout_passes=False),
    scratch_types=[
        pltpu.VMEM((2, PB * G), jnp.int32),
        pltpu.VMEM((2, PB * G), jnp.int32),
        pltpu.VMEM((2, PB * G), jnp.float32),
        [[pltpu.VMEM((FLCAP,), jnp.int32),
          pltpu.VMEM((FLCAP,), jnp.int32),
          pltpu.VMEM((FLCAP,), jnp.int32)] for _ in range(2)],
        pltpu.VMEM((16,), jnp.int32),
        pltpu.SemaphoreType.DMA,
        pltpu.SemaphoreType.DMA,
    ],
)


def _propagate_body(feat, pedge, cnt, out,
                    edgev, dstl, buf, acc, cbuf, lsem, gsem, ssem):
    c = lax.axis_index("c")
    s = lax.axis_index("s")
    off = c * HALF

    zero16 = jnp.zeros((16,), jnp.float32)
    for i in range(G):
        buf[0, i, pl.ds(0, 16)] = zero16
        buf[0, i, pl.ds(16, 16)] = zero16
    zbase = s * ZPT
    for z in range(ZPT // G):
        pltpu.sync_copy(buf.at[0], acc.at[pl.ds(zbase + z * G, G)])
    rem = ZPT - (ZPT // G) * G
    if rem:
        pltpu.sync_copy(buf.at[0, pl.ds(0, rem)],
                        acc.at[pl.ds(zbase + (ZPT // G) * G, rem)])
    plsc.subcore_barrier()

    for r_ofs in (0, NS):
        r = s + r_ofs

        pltpu.sync_copy(cnt.at[c, r], cbuf)
        ng = cbuf[pl.ds(0, 16)][0]
        nb = lax.shift_right_logical(ng, 2)

        def fire_lin(b, slot):
            be = pl.multiple_of(b * SB * G, SB * G)
            pltpu.async_copy(pedge.at[c, r, slice(None), pl.ds(be, SB * G)],
                             edgev.at[slot], lsem)

        def wait_lin(slot):
            pltpu.make_async_copy(pedge.at[c, r, slice(None), pl.ds(0, SB * G)],
                                  edgev.at[slot], lsem).wait()

        @pl.when(nb > 0)
        def _region(r=r, ng=ng, nb=nb, fire_lin=fire_lin, wait_lin=wait_lin):
            fire_lin(0, 0)

            def block(b, carry):
                slot = lax.rem(b, 2)
                wait_lin(slot)

                @pl.when(b + 1 < nb)
                def _next_lin():
                    fire_lin(b + 1, 1 - slot)

                for gi in range(SB):
                    for q in range(8):
                        d16 = edgev[slot, 1, pl.ds(gi * G + q * 16, 16)] - off
                        okm = (d16 >= 0) & (d16 < HALF)
                        dstl_p = [dstl[gi], dstl[SB + gi]]
                        for p in range(2):
                            @pl.when(slot == p)
                            def _wr(p=p, d16=d16, okm=okm, gi=gi, q=q):
                                dstl_p[p][pl.ds(q * 16, 16)] = jnp.where(okm, d16, HALF)

                gdescs = []
                for gi in range(SB):
                    @pl.when(b > 0)
                    def _drain_prev(gi=gi):
                        pltpu.make_async_copy(buf.at[gi], acc.at[dstl[gi]], ssem).wait()
                    gdescs.append(pltpu.async_copy(
                        feat.at[edgev.at[slot, 0, pl.ds(gi * G, G)]], buf.at[gi], gsem))

                for gi in range(SB):
                    gdescs[gi].wait()
                    for q in range(8):
                        v16 = plsc.bitcast(
                            edgev[slot, 2, pl.ds(gi * G + q * 16, 16)], jnp.float32)
                        for e in range(16):
                            v = _bcast_lane(v16, e)
                            rr = q * 16 + e
                            buf[gi, rr, pl.ds(0, 16)] = buf[gi, rr, pl.ds(0, 16)] * v
                            buf[gi, rr, pl.ds(16, 16)] = buf[gi, rr, pl.ds(16, 16)] * v
                    for p in range(2):
                        @pl.when(slot == p)
                        def _sc(p=p, gi=gi):
                            pltpu.async_copy(buf.at[gi], acc.at[dstl[p * SB + gi]],
                                             ssem, add=True)
                return carry

            lax.fori_loop(0, nb, block, 0)
            for gi in range(SB):
                pltpu.make_async_copy(buf.at[gi], acc.at[dstl[gi]], ssem).wait()

    plsc.subcore_barrier()

    ob = s * ZPT

    @pl.when(s < NS - 1)
    def _copy_full():
        pltpu.sync_copy(acc.at[pl.ds(ob, ZPT)], out.at[pl.ds(off + ob, ZPT)])

    @pl.when(s == NS - 1)
    def _copy_tail():
        pltpu.sync_copy(acc.at[pl.ds((NS - 1) * ZPT, HALF - (NS - 1) * ZPT)],
                        out.at[pl.ds(off + (NS - 1) * ZPT, HALF - (NS - 1) * ZPT)])


_propagate = pl.kernel(
    _propagate_body,
    out_type=jax.ShapeDtypeStruct((N_NODES, EMB), jnp.float32),
    mesh=_mesh,
    compiler_params=pltpu.CompilerParams(use_tc_tiling_on_sc=False,
                                         needs_layout_passes=False),
    scratch_types=[
        pltpu.VMEM((2, 3, SB * G), jnp.int32),
        [pltpu.VMEM((G,), jnp.int32) for _ in range(2 * SB)],
        pltpu.VMEM((SB, G, EMB), jnp.float32),
        pltpu.VMEM_SHARED((ACC_ROWS, EMB), jnp.float32),
        pltpu.VMEM((16,), jnp.int32),
        pltpu.SemaphoreType.DMA,
        pltpu.SemaphoreType.DMA,
        pltpu.SemaphoreType.DMA,
    ],
)


def _finalize_body(f0, f1, f2, f3, uidx, pidx, nidx, ue, pe, ne,
                   iv, iv2, ba, bb, bc, bd):
    c = lax.axis_index("c")
    s = lax.axis_index("s")
    w = s * NC + c

    for idx_hbm, out_hbm, base_off in ((uidx, ue, 0), (pidx, pe, HALF), (nidx, ne, HALF)):
        pltpu.sync_copy(idx_hbm.at[w], iv)
        for q in range(8):
            j0 = q * 16
            iv2[pl.ds(j0, 16)] = iv[0, pl.ds(j0, 16)] + base_off
        for arr, b in ((f0, ba), (f1, bb), (f2, bc), (f3, bd)):
            pltpu.sync_copy(arr.at[iv2], b)

        def mean_rows(q, carry):
            for jj in range(16):
                for h in range(2):
                    sl = pl.ds(16 * h, 16)
                    m = (ba[q * 16 + jj, sl] + bb[q * 16 + jj, sl]
                         + bc[q * 16 + jj, sl] + bd[q * 16 + jj, sl]) * 0.25
                    ba[q * 16 + jj, sl] = m
            return carry

        lax.fori_loop(0, G // 16, mean_rows, 0)
        pltpu.sync_copy(ba, out_hbm.at[pl.ds(w * G, G)])


_finalize = pl.kernel(
    _finalize_body,
    out_type=(
        jax.ShapeDtypeStruct((BATCH, EMB), jnp.float32),
        jax.ShapeDtypeStruct((BATCH, EMB), jnp.float32),
        jax.ShapeDtypeStruct((BATCH, EMB), jnp.float32),
    ),
    mesh=_mesh,
    compiler_params=pltpu.CompilerParams(use_tc_tiling_on_sc=False),
    scratch_types=[
        pltpu.VMEM((1, G), jnp.int32),
        pltpu.VMEM((G,), jnp.int32),
        pltpu.VMEM((G, EMB), jnp.float32),
        pltpu.VMEM((G, EMB), jnp.float32),
        pltpu.VMEM((G, EMB), jnp.float32),
        pltpu.VMEM((G, EMB), jnp.float32),
    ],
)


def kernel(user, pos_item, neg_item, user_table, item_table,
           adj_rows, adj_cols, adj_vals):
    node0 = jnp.concatenate([user_table, item_table], axis=0)
    cols = adj_cols.astype(jnp.int32)
    rows = adj_rows.astype(jnp.int32)
    vals = adj_vals

    pedge, cnt = _partition(cols, rows, vals)

    feats = [node0]
    f = node0
    for _ in range(LAYERS):
        f = _propagate(f, pedge, cnt)
        feats.append(f)

    u2 = user.astype(jnp.int32).reshape(-1, 1, G)
    p2 = pos_item.astype(jnp.int32).reshape(-1, 1, G)
    n2 = neg_item.astype(jnp.int32).reshape(-1, 1, G)
    return _finalize(feats[0], feats[1], feats[2], feats[3], u2, p2, n2)


# speedup vs baseline: 1.1766x; 1.1766x over previous
import jax
import jax.numpy as jnp
from jax import lax
from jax.experimental import pallas as pl
from jax.experimental.pallas import tpu as pltpu
from jax.experimental.pallas import tpu_sc as plsc

N_USERS = 50000
N_ITEMS = 50000
N_NODES = 100000
EMB = 32
LAYERS = 3
BATCH = 4096

NC = 2
NS = 16
NW = NC * NS
G = 128
SB = 4
HALF = 50000
ACC_ROWS = 50176
ZPT = ACC_ROWS // NS

TE = 50000
PB = 8
PNB = TE // (PB * G)
TAILE = TE - PNB * PB * G
FLCAP = 1280
CAP = TE + 1024

_mesh = plsc.VectorSubcoreMesh(core_axis_name="c", subcore_axis_name="s")

_GDN = lax.GatherDimensionNumbers(
    offset_dims=(), collapsed_slice_dims=(0,), start_index_map=(0,))


def _bcast_lane(v16, e):
    idx = jnp.full((16, 1), e, jnp.int32)
    return lax.gather(v16, idx, _GDN, (1,),
                      mode=lax.GatherScatterMode.PROMISE_IN_BOUNDS)


def _partition_body(cols, rows, vals, pcol, prow, pval, cnt,
                    icol, irow, ival, bufs, cbuf, lsem, fsem):
    c = lax.axis_index("c")
    s = lax.axis_index("s")
    w = s * NC + c
    ebase = w * TE

    (bc0, br0, bv0), (bc1, br1, bv1) = bufs

    def fire_lin(b, slot):
        be = ebase + b * PB * G
        pltpu.async_copy(cols.at[pl.ds(be, PB * G)], icol.at[slot], lsem)
        pltpu.async_copy(rows.at[pl.ds(be, PB * G)], irow.at[slot], lsem)
        pltpu.async_copy(vals.at[pl.ds(be, PB * G)], ival.at[slot], lsem)

    def wait_lin(slot):
        pltpu.make_async_copy(cols.at[pl.ds(0, PB * G)], icol.at[slot], lsem).wait()
        pltpu.make_async_copy(rows.at[pl.ds(0, PB * G)], irow.at[slot], lsem).wait()
        pltpu.make_async_copy(vals.at[pl.ds(0, PB * G)], ival.at[slot], lsem).wait()

    fire_lin(0, 0)

    def flush(h, bc, br, bv, cur, hb):
        nf = lax.shift_right_logical(cur, 7)
        hb = pl.multiple_of(hb, G)

        def fcopy(g, carry):
            o = pl.multiple_of(g * G, G)
            pltpu.async_copy(bc.at[pl.ds(o, G)], pcol.at[h, w, pl.ds(hb + o, G)], fsem)
            pltpu.async_copy(br.at[pl.ds(o, G)], prow.at[h, w, pl.ds(hb + o, G)], fsem)
            pltpu.async_copy(bv.at[pl.ds(o, G)], pval.at[h, w, pl.ds(hb + o, G)], fsem)
            return carry

        lax.fori_loop(0, nf, fcopy, 0)

        def fdrain(g, carry):
            o = pl.multiple_of(g * G, G)
            pltpu.make_async_copy(bc.at[pl.ds(o, G)], pcol.at[h, w, pl.ds(hb + o, G)], fsem).wait()
            pltpu.make_async_copy(br.at[pl.ds(o, G)], prow.at[h, w, pl.ds(hb + o, G)], fsem).wait()
            pltpu.make_async_copy(bv.at[pl.ds(o, G)], pval.at[h, w, pl.ds(hb + o, G)], fsem).wait()
            return carry

        lax.fori_loop(0, nf, fdrain, 0)

        fb = nf * G
        for t in range(G // 16):
            tc = bc[pl.ds(fb + t * 16, 16)]
            tr = br[pl.ds(fb + t * 16, 16)]
            tv = bv[pl.ds(fb + t * 16, 16)]
            bc[pl.ds(t * 16, 16)] = tc
            br[pl.ds(t * 16, 16)] = tr
            bv[pl.ds(t * 16, 16)] = tv
        return cur - fb, hb + fb

    def route(slot, q, cur0, cur1):
        c16 = icol[slot, pl.ds(q * 16, 16)]
        r16 = irow[slot, pl.ds(q * 16, 16)]
        v16 = ival[slot, pl.ds(q * 16, 16)]
        m0 = r16 < HALF
        m1 = jnp.logical_not(m0)
        plsc.store_compressed(bc0.at[pl.ds(cur0, 16)], c16, mask=m0)
        plsc.store_compressed(br0.at[pl.ds(cur0, 16)], r16, mask=m0)
        plsc.store_compressed(bv0.at[pl.ds(cur0, 16)], v16, mask=m0)
        plsc.store_compressed(bc1.at[pl.ds(cur1, 16)], c16, mask=m1)
        plsc.store_compressed(br1.at[pl.ds(cur1, 16)], r16, mask=m1)
        plsc.store_compressed(bv1.at[pl.ds(cur1, 16)], v16, mask=m1)
        n0 = plsc.all_reduce_population_count(m0)[0]
        return cur0 + n0, cur1 + (16 - n0)

    def block(b, carry):
        cur0, cur1, hb0, hb1 = carry
        slot = lax.rem(b, 2)
        wait_lin(slot)

        @pl.when(b + 1 < PNB)
        def _next():
            fire_lin(b + 1, 1 - slot)

        for q in range(PB * G // 16):
            cur0, cur1 = route(slot, q, cur0, cur1)

        cur0, hb0 = flush(0, bc0, br0, bv0, cur0, hb0)
        cur1, hb1 = flush(1, bc1, br1, bv1, cur1, hb1)
        return cur0, cur1, hb0, hb1

    cur0, cur1, hb0, hb1 = lax.fori_loop(
        0, PNB, block, (jnp.int32(0), jnp.int32(0), jnp.int32(0), jnp.int32(0)))

    pltpu.sync_copy(cols.at[pl.ds(ebase + PNB * PB * G, TAILE)],
                    icol.at[0, pl.ds(0, TAILE)])
    pltpu.sync_copy(rows.at[pl.ds(ebase + PNB * PB * G, TAILE)],
                    irow.at[0, pl.ds(0, TAILE)])
    pltpu.sync_copy(vals.at[pl.ds(ebase + PNB * PB * G, TAILE)],
                    ival.at[0, pl.ds(0, TAILE)])
    for q in range(TAILE // 16):
        cur0, cur1 = route(0, q, cur0, cur1)
    cur0, hb0 = flush(0, bc0, br0, bv0, cur0, hb0)
    cur1, hb1 = flush(1, bc1, br1, bv1, cur1, hb1)

    zi = jnp.zeros((16,), jnp.int32)
    zf = jnp.zeros((16,), jnp.float32)
    for (bc, br, bv, cur) in ((bc0, br0, bv0, cur0), (bc1, br1, bv1, cur1)):
        for t in range(512 // 16):
            bc[pl.ds(cur + t * 16, 16)] = zi
            br[pl.ds(cur + t * 16, 16)] = zi
            bv[pl.ds(cur + t * 16, 16)] = zf
    for h, (bc, br, bv, cur, hb) in enumerate(
            ((bc0, br0, bv0, cur0, hb0), (bc1, br1, bv1, cur1, hb1))):
        total = hb + cur
        totp = jnp.bitwise_and(total + 511, -512)
        curp = totp - hb
        flush(h, bc, br, bv, curp, hb)
        ng = lax.shift_right_logical(totp, 7)
        cbuf[pl.ds(0, 16)] = jnp.full((16,), ng, jnp.int32)
        pltpu.sync_copy(cbuf, cnt.at[h, w])


_partition = pl.kernel(
    _partition_body,
    out_type=(
        jax.ShapeDtypeStruct((2, NW, CAP), jnp.int32),
        jax.ShapeDtypeStruct((2, NW, CAP), jnp.int32),
        jax.ShapeDtypeStruct((2, NW, CAP), jnp.float32),
        jax.ShapeDtypeStruct((2, NW, 16), jnp.int32),
    ),
    mesh=_mesh,
    compiler_params=pltpu.CompilerParams(use_tc_tiling_on_sc=False,
                                         needs_layout_passes=False),
    scratch_types=[
        pltpu.VMEM((2, PB * G), jnp.int32),
        pltpu.VMEM((2, PB * G), jnp.int32),
        pltpu.VMEM((2, PB * G), jnp.float32),
        [[pltpu.VMEM((FLCAP,), jnp.int32),
          pltpu.VMEM((FLCAP,), jnp.int32),
          pltpu.VMEM((FLCAP,), jnp.float32)] for _ in range(2)],
        pltpu.VMEM((16,), jnp.int32),
        pltpu.SemaphoreType.DMA,
        pltpu.SemaphoreType.DMA,
    ],
)


def _propagate_body(feat, pcol, prow, pval, cnt, out,
                    colv, rowv, valv, dstl, buf, acc, cbuf, lsem, gsem, ssem):
    c = lax.axis_index("c")
    s = lax.axis_index("s")
    off = c * HALF

    zero16 = jnp.zeros((16,), jnp.float32)
    for i in range(G):
        buf[0, i, pl.ds(0, 16)] = zero16
        buf[0, i, pl.ds(16, 16)] = zero16
    zbase = s * ZPT
    for z in range(ZPT // G):
        pltpu.sync_copy(buf.at[0], acc.at[pl.ds(zbase + z * G, G)])
    rem = ZPT - (ZPT // G) * G
    if rem:
        pltpu.sync_copy(buf.at[0, pl.ds(0, rem)],
                        acc.at[pl.ds(zbase + (ZPT // G) * G, rem)])
    plsc.subcore_barrier()

    for r_ofs in (0, NS):
        r = s + r_ofs

        pltpu.sync_copy(cnt.at[c, r], cbuf)
        ng = cbuf[pl.ds(0, 16)][0]
        nb = lax.shift_right_logical(ng, 2)

        def fire_lin(b, slot):
            be = pl.multiple_of(b * SB * G, SB * G)
            pltpu.async_copy(pcol.at[c, r, pl.ds(be, SB * G)], colv.at[slot], lsem)
            pltpu.async_copy(prow.at[c, r, pl.ds(be, SB * G)], rowv.at[slot], lsem)
            pltpu.async_copy(pval.at[c, r, pl.ds(be, SB * G)], valv.at[slot], lsem)

        def wait_lin(slot):
            pltpu.make_async_copy(pcol.at[c, r, pl.ds(0, SB * G)], colv.at[slot], lsem).wait()
            pltpu.make_async_copy(prow.at[c, r, pl.ds(0, SB * G)], rowv.at[slot], lsem).wait()
            pltpu.make_async_copy(pval.at[c, r, pl.ds(0, SB * G)], valv.at[slot], lsem).wait()

        @pl.when(nb > 0)
        def _region(r=r, ng=ng, nb=nb, fire_lin=fire_lin, wait_lin=wait_lin):
            fire_lin(0, 0)

            def block(b, carry):
                slot = lax.rem(b, 2)
                wait_lin(slot)

                @pl.when(b + 1 < nb)
                def _next_lin():
                    fire_lin(b + 1, 1 - slot)

                for gi in range(SB):
                    for q in range(8):
                        d16 = rowv[slot, pl.ds(gi * G + q * 16, 16)] - off
                        okm = (d16 >= 0) & (d16 < HALF)
                        dstl_p = [dstl[gi], dstl[SB + gi]]
                        for p in range(2):
                            @pl.when(slot == p)
                            def _wr(p=p, d16=d16, okm=okm, gi=gi, q=q):
                                dstl_p[p][pl.ds(q * 16, 16)] = jnp.where(okm, d16, HALF)

                gdescs = []
                for gi in range(SB):
                    @pl.when(b > 0)
                    def _drain_prev(gi=gi):
                        pltpu.make_async_copy(buf.at[gi], acc.at[dstl[gi]], ssem).wait()
                    gdescs.append(pltpu.async_copy(
                        feat.at[colv.at[slot, pl.ds(gi * G, G)]], buf.at[gi], gsem))

                for gi in range(SB):
                    gdescs[gi].wait()
                    for q in range(8):
                        v16 = valv[slot, pl.ds(gi * G + q * 16, 16)]
                        for e in range(16):
                            v = _bcast_lane(v16, e)
                            rr = q * 16 + e
                            buf[gi, rr, pl.ds(0, 16)] = buf[gi, rr, pl.ds(0, 16)] * v
                            buf[gi, rr, pl.ds(16, 16)] = buf[gi, rr, pl.ds(16, 16)] * v
                    for p in range(2):
                        @pl.when(slot == p)
                        def _sc(p=p, gi=gi):
                            pltpu.async_copy(buf.at[gi], acc.at[dstl[p * SB + gi]],
                                             ssem, add=True)
                return carry

            lax.fori_loop(0, nb, block, 0)
            for gi in range(SB):
                pltpu.make_async_copy(buf.at[gi], acc.at[dstl[gi]], ssem).wait()

    plsc.subcore_barrier()

    ob = s * ZPT

    @pl.when(s < NS - 1)
    def _copy_full():
        pltpu.sync_copy(acc.at[pl.ds(ob, ZPT)], out.at[pl.ds(off + ob, ZPT)])

    @pl.when(s == NS - 1)
    def _copy_tail():
        pltpu.sync_copy(acc.at[pl.ds((NS - 1) * ZPT, HALF - (NS - 1) * ZPT)],
                        out.at[pl.ds(off + (NS - 1) * ZPT, HALF - (NS - 1) * ZPT)])


_propagate = pl.kernel(
    _propagate_body,
    out_type=jax.ShapeDtypeStruct((N_NODES, EMB), jnp.float32),
    mesh=_mesh,
    compiler_params=pltpu.CompilerParams(use_tc_tiling_on_sc=False),
    scratch_types=[
        pltpu.VMEM((2, SB * G), jnp.int32),
        pltpu.VMEM((2, SB * G), jnp.int32),
        pltpu.VMEM((2, SB * G), jnp.float32),
        [pltpu.VMEM((G,), jnp.int32) for _ in range(2 * SB)],
        pltpu.VMEM((SB, G, EMB), jnp.float32),
        pltpu.VMEM_SHARED((ACC_ROWS, EMB), jnp.float32),
        pltpu.VMEM((16,), jnp.int32),
        pltpu.SemaphoreType.DMA,
        pltpu.SemaphoreType.DMA,
        pltpu.SemaphoreType.DMA,
    ],
)


def _finalize_body(f0, f1, f2, f3, uidx, pidx, nidx, ue, pe, ne,
                   iv, iv2, ba, bb, bc, bd):
    c = lax.axis_index("c")
    s = lax.axis_index("s")
    w = s * NC + c

    for idx_hbm, out_hbm, base_off in ((uidx, ue, 0), (pidx, pe, HALF), (nidx, ne, HALF)):
        pltpu.sync_copy(idx_hbm.at[w], iv)
        for q in range(8):
            j0 = q * 16
            iv2[pl.ds(j0, 16)] = iv[0, pl.ds(j0, 16)] + base_off
        for arr, b in ((f0, ba), (f1, bb), (f2, bc), (f3, bd)):
            pltpu.sync_copy(arr.at[iv2], b)

        def mean_rows(q, carry):
            for jj in range(16):
                for h in range(2):
                    sl = pl.ds(16 * h, 16)
                    m = (ba[q * 16 + jj, sl] + bb[q * 16 + jj, sl]
                         + bc[q * 16 + jj, sl] + bd[q * 16 + jj, sl]) * 0.25
                    ba[q * 16 + jj, sl] = m
            return carry

        lax.fori_loop(0, G // 16, mean_rows, 0)
        pltpu.sync_copy(ba, out_hbm.at[pl.ds(w * G, G)])


_finalize = pl.kernel(
    _finalize_body,
    out_type=(
        jax.ShapeDtypeStruct((BATCH, EMB), jnp.float32),
        jax.ShapeDtypeStruct((BATCH, EMB), jnp.float32),
        jax.ShapeDtypeStruct((BATCH, EMB), jnp.float32),
    ),
    mesh=_mesh,
    compiler_params=pltpu.CompilerParams(use_tc_tiling_on_sc=False),
    scratch_types=[
        pltpu.VMEM((1, G), jnp.int32),
        pltpu.VMEM((G,), jnp.int32),
        pltpu.VMEM((G, EMB), jnp.float32),
        pltpu.VMEM((G, EMB), jnp.float32),
        pltpu.VMEM((G, EMB), jnp.float32),
        pltpu.VMEM((G, EMB), jnp.float32),
    ],
)


def kernel(user, pos_item, neg_item, user_table, item_table,
           adj_rows, adj_cols, adj_vals):
    node0 = jnp.concatenate([user_table, item_table], axis=0)
    cols = adj_cols.astype(jnp.int32)
    rows = adj_rows.astype(jnp.int32)
    vals = adj_vals

    pcol, prow, pval, cnt = _partition(cols, rows, vals)

    feats = [node0]
    f = node0
    for _ in range(LAYERS):
        f = _propagate(f, pcol, prow, pval, cnt)
        feats.append(f)

    u2 = user.astype(jnp.int32).reshape(-1, 1, G)
    p2 = pos_item.astype(jnp.int32).reshape(-1, 1, G)
    n2 = neg_item.astype(jnp.int32).reshape(-1, 1, G)
    return _finalize(feats[0], feats[1], feats[2], feats[3], u2, p2, n2)
